# Initial kernel scaffold; baseline (speedup 1.0000x reference)
import jax
import jax.numpy as jnp
from jax.experimental import pallas as pl


def kernel(sample, entity_embedding, relation_embedding):
    raise NotImplementedError("write your pallas kernel here")


# speedup vs baseline: 2.2067x; 2.2067x over previous
import functools

import jax
import jax.numpy as jnp
from jax import lax
from jax.experimental import pallas as pl
from jax.experimental.pallas import tpu as pltpu
from jax.experimental.pallas import tpu_sc as plsc

_D = 64
_B = 16384
_GAMMA = 12.0
_NC = 2
_NS = 16
_NW = _NC * _NS
_BPW = _B // _NW
_IC = 128
_NCHUNK = _BPW // _IC
_L = 16


def _tec_body(hidx, ridx, tidx, ent, rel, out,
              hix, rix, tix, h_v, r_v, t_v, cs_v, o_v, sem_a, sem_b):
    wid = lax.axis_index("s") * _NC + lax.axis_index("c")
    ibase = wid * _NCHUNK
    base = wid * _BPW

    pltpu.sync_copy(hidx.at[pl.ds(ibase, _NCHUNK)], hix)
    pltpu.sync_copy(ridx.at[pl.ds(ibase, _NCHUNK)], rix)
    pltpu.sync_copy(tidx.at[pl.ds(ibase, _NCHUNK)], tix)

    copies = []
    for j in range(_NCHUNK):
        rows = pl.ds(j * _IC, _IC)
        copies.append(pltpu.async_copy(ent.at[hix.at[j]], h_v.at[rows], sem_a))
        copies.append(pltpu.async_copy(rel.at[rix.at[j]], r_v.at[rows], sem_a))
        copies.append(pltpu.async_copy(ent.at[tix.at[j]], t_v.at[rows], sem_b))
    for c in copies:
        c.wait()

    row_ids = lax.iota(jnp.int32, _L)

    def body(g, carry):
        for k in range(_L):
            s = g * _L + k
            acc = None
            for c in range(_D // _L):
                cols = pl.ds(c * _L, _L)
                a = jnp.abs(h_v[s, cols] + r_v[s, cols] - t_v[s, cols])
                acc = a if acc is None else acc + a
            col_k = jnp.full((_L,), k, jnp.int32)
            plsc.store_scatter(cs_v, [row_ids, col_k], acc)
        sums = None
        for j in range(_L):
            rowv = cs_v[j, :]
            sums = rowv if sums is None else sums + rowv
        o_v[pl.ds(g * _L, _L)] = _GAMMA - sums
        return carry

    lax.fori_loop(0, _BPW // _L, body, 0)

    pltpu.sync_copy(o_v, out.at[pl.ds(base, _BPW)])


@functools.cache
def _build():
    mesh = plsc.VectorSubcoreMesh(
        core_axis_name="c", subcore_axis_name="s",
        num_cores=_NC, num_subcores=_NS)
    return pl.kernel(
        _tec_body,
        out_type=jax.ShapeDtypeStruct((_B,), jnp.float32),
        mesh=mesh,
        compiler_params=pltpu.CompilerParams(
            needs_layout_passes=False, use_tc_tiling_on_sc=False),
        scratch_types=[
            pltpu.VMEM((_NCHUNK, _IC), jnp.int32),
            pltpu.VMEM((_NCHUNK, _IC), jnp.int32),
            pltpu.VMEM((_NCHUNK, _IC), jnp.int32),
            pltpu.VMEM((_BPW, _D), jnp.float32),
            pltpu.VMEM((_BPW, _D), jnp.float32),
            pltpu.VMEM((_BPW, _D), jnp.float32),
            pltpu.VMEM((_L, _L), jnp.float32),
            pltpu.VMEM((_BPW,), jnp.float32),
            pltpu.SemaphoreType.DMA,
            pltpu.SemaphoreType.DMA,
        ],
    )


@jax.jit
def kernel(sample, entity_embedding, relation_embedding):
    sample = sample.astype(jnp.int32)
    hidx = sample[:, 0].reshape(_NW * _NCHUNK, _IC)
    ridx = sample[:, 1].reshape(_NW * _NCHUNK, _IC)
    tidx = sample[:, 2].reshape(_NW * _NCHUNK, _IC)
    out = _build()(hidx, ridx, tidx, entity_embedding, relation_embedding)
    return out.reshape(_B, 1)


# speedup vs baseline: 2.4189x; 1.0961x over previous
import functools

import jax
import jax.numpy as jnp
from jax import lax
from jax.experimental import pallas as pl
from jax.experimental.pallas import tpu as pltpu
from jax.experimental.pallas import tpu_sc as plsc

_D = 64
_B = 16384
_GAMMA = 12.0
_NC = 2
_NS = 16
_NW = _NC * _NS
_BPW = _B // _NW
_IC = 128
_NCHUNK = _BPW // _IC
_L = 16


def _tec_body(idx_hbm, ent, rel, out, ix, h_v, r_v, t_v, cs_v, o_v, *sems):
    wid = lax.axis_index("s") * _NC + lax.axis_index("c")
    base = wid * _BPW

    pltpu.sync_copy(idx_hbm.at[wid], ix)

    copies = []
    for j in range(_NCHUNK):
        rows = pl.ds(j * _IC, _IC)
        copies.append(pltpu.async_copy(
            ent.at[ix.at[0, j]], h_v.at[rows], sems[3 * j]))
        copies.append(pltpu.async_copy(
            rel.at[ix.at[1, j]], r_v.at[rows], sems[3 * j + 1]))
        copies.append(pltpu.async_copy(
            ent.at[ix.at[2, j]], t_v.at[rows], sems[3 * j + 2]))

    row_ids = lax.iota(jnp.int32, _L)

    def group(g):
        for k in range(_L):
            s = g * _L + k
            acc = None
            for c in range(_D // _L):
                cols = pl.ds(c * _L, _L)
                a = jnp.abs(h_v[s, cols] + r_v[s, cols] - t_v[s, cols])
                acc = a if acc is None else acc + a
            col_k = jnp.full((_L,), k, jnp.int32)
            plsc.store_scatter(cs_v, [row_ids, col_k], acc)
        sums = None
        for j in range(_L):
            rowv = cs_v[j, :]
            sums = rowv if sums is None else sums + rowv
        o_v[pl.ds(g * _L, _L)] = _GAMMA - sums

    gpc = _IC // _L
    for j in range(_NCHUNK):
        for c in copies[3 * j:3 * j + 3]:
            c.wait()

        def body(i, carry):
            group(j * gpc + i)
            return carry

        lax.fori_loop(0, gpc, body, 0)

    pltpu.sync_copy(o_v, out.at[pl.ds(base, _BPW)])


@functools.cache
def _build():
    mesh = plsc.VectorSubcoreMesh(
        core_axis_name="c", subcore_axis_name="s",
        num_cores=_NC, num_subcores=_NS)
    return pl.kernel(
        _tec_body,
        out_type=jax.ShapeDtypeStruct((_B,), jnp.float32),
        mesh=mesh,
        compiler_params=pltpu.CompilerParams(
            needs_layout_passes=False, use_tc_tiling_on_sc=False),
        scratch_types=[
            pltpu.VMEM((3, _NCHUNK, _IC), jnp.int32),
            pltpu.VMEM((_BPW, _D), jnp.float32),
            pltpu.VMEM((_BPW, _D), jnp.float32),
            pltpu.VMEM((_BPW, _D), jnp.float32),
            pltpu.VMEM((_L, _L), jnp.float32),
            pltpu.VMEM((_BPW,), jnp.float32),
        ] + [pltpu.SemaphoreType.DMA] * (3 * _NCHUNK),
    )


@jax.jit
def kernel(sample, entity_embedding, relation_embedding):
    sample = sample.astype(jnp.int32)
    idx = sample.T.reshape(3, _NW, _NCHUNK, _IC).transpose(1, 0, 2, 3)
    out = _build()(idx, entity_embedding, relation_embedding)
    return out.reshape(_B, 1)


# speedup vs baseline: 2.4194x; 1.0002x over previous
import functools

import jax
import jax.numpy as jnp
from jax import lax
from jax.experimental import pallas as pl
from jax.experimental.pallas import tpu as pltpu
from jax.experimental.pallas import tpu_sc as plsc

_D = 64
_B = 16384
_GAMMA = 12.0
_NC = 2
_NS = 16
_NW = _NC * _NS
_BPW = _B // _NW
_IC = 128
_NCHUNK = _BPW // _IC
_L = 16


def _tec_body(idx_hbm, ent, rel, out, ix, h_v, t_v, cs_v, o_v, *sems):
    wid = lax.axis_index("s") * _NC + lax.axis_index("c")
    base = wid * _BPW

    pltpu.sync_copy(idx_hbm.at[wid], ix)

    h_copies, t_copies, r_copies = [], [], []
    for j in range(_NCHUNK):
        rows = pl.ds(j * _IC, _IC)
        h_copies.append(pltpu.async_copy(
            ent.at[ix.at[0, j]], h_v.at[rows], sems[3 * j]))
        t_copies.append(pltpu.async_copy(
            ent.at[ix.at[2, j]], t_v.at[rows], sems[3 * j + 2]))
    for j in range(_NCHUNK):
        rows = pl.ds(j * _IC, _IC)
        h_copies[j].wait()
        r_copies.append(pltpu.async_copy(
            rel.at[ix.at[1, j]], h_v.at[rows], sems[3 * j + 1], add=True))

    row_ids = lax.iota(jnp.int32, _L)

    def group(g):
        for k in range(_L):
            s = g * _L + k
            acc = None
            for c in range(_D // _L):
                cols = pl.ds(c * _L, _L)
                a = jnp.abs(h_v[s, cols] - t_v[s, cols])
                acc = a if acc is None else acc + a
            col_k = jnp.full((_L,), k, jnp.int32)
            plsc.store_scatter(cs_v, [row_ids, col_k], acc)
        sums = None
        for j in range(_L):
            rowv = cs_v[j, :]
            sums = rowv if sums is None else sums + rowv
        o_v[pl.ds(g * _L, _L)] = _GAMMA - sums

    gpc = _IC // _L
    for j in range(_NCHUNK):
        r_copies[j].wait()
        t_copies[j].wait()

        def body(i, carry):
            group(j * gpc + i)
            return carry

        lax.fori_loop(0, gpc, body, 0)

    pltpu.sync_copy(o_v, out.at[pl.ds(base, _BPW)])


@functools.cache
def _build():
    mesh = plsc.VectorSubcoreMesh(
        core_axis_name="c", subcore_axis_name="s",
        num_cores=_NC, num_subcores=_NS)
    return pl.kernel(
        _tec_body,
        out_type=jax.ShapeDtypeStruct((_B,), jnp.float32),
        mesh=mesh,
        compiler_params=pltpu.CompilerParams(
            needs_layout_passes=False, use_tc_tiling_on_sc=False),
        scratch_types=[
            pltpu.VMEM((3, _NCHUNK, _IC), jnp.int32),
            pltpu.VMEM((_BPW, _D), jnp.float32),
            pltpu.VMEM((_BPW, _D), jnp.float32),
            pltpu.VMEM((_L, _L), jnp.float32),
            pltpu.VMEM((_BPW,), jnp.float32),
        ] + [pltpu.SemaphoreType.DMA] * (3 * _NCHUNK),
    )


@jax.jit
def kernel(sample, entity_embedding, relation_embedding):
    sample = sample.astype(jnp.int32)
    idx = sample.T.reshape(3, _NW, _NCHUNK, _IC).transpose(1, 0, 2, 3)
    out = _build()(idx, entity_embedding, relation_embedding)
    return out.reshape(_B, 1)
